# baseline (speedup 1.0000x reference)
import functools

import jax
import jax.numpy as jnp
from jax import lax
from jax.experimental import pallas as pl
from jax.experimental.pallas import tpu as pltpu
from jax.experimental.pallas import tpu_sc as plsc

NUM_CORES = 2
NUM_SUBCORES = 16
NUM_WORKERS = NUM_CORES * NUM_SUBCORES
CHUNK = 128
GPB = 2
NBUF = 2


@functools.partial(jax.jit, static_argnums=(2, 3))
def _embedding_gather(x_flat, W, B, D):
  b_per_w = B // NUM_WORKERS
  n_super = b_per_w // (CHUNK * GPB)
  n_groups = n_super // NBUF
  mesh = plsc.VectorSubcoreMesh(
      core_axis_name="c", subcore_axis_name="s",
      num_cores=NUM_CORES, num_subcores=NUM_SUBCORES)

  @functools.partial(
      pl.kernel,
      out_type=jax.ShapeDtypeStruct((B, D), jnp.float32),
      mesh=mesh,
      scratch_types=(
          [pltpu.VMEM((b_per_w,), jnp.int32)]
          + [pltpu.VMEM((GPB * CHUNK, D), jnp.float32) for _ in range(NBUF)]
          + [pltpu.SemaphoreType.DMA for _ in range(2 * NBUF)]
      ),
  )
  def k(table_hbm, idx_hbm, out_hbm, idx_all, *bufs_and_sems):
    rows = bufs_and_sems[:NBUF]
    sg = bufs_and_sems[NBUF:2 * NBUF]
    sw = bufs_and_sems[2 * NBUF:3 * NBUF]
    wid = lax.axis_index("s") * NUM_CORES + lax.axis_index("c")
    base = wid * b_per_w

    pltpu.sync_copy(idx_hbm.at[pl.ds(base, b_per_w)], idx_all)

    def start_gathers(i, b):
      for h in range(GPB):
        pltpu.async_copy(
            table_hbm.at[idx_all.at[pl.ds((i * GPB + h) * CHUNK, CHUNK)]],
            rows[b].at[pl.ds(h * CHUNK, CHUNK)], sg[b])

    def wait_gathers(b):
      for h in range(GPB):
        pltpu.make_async_copy(
            table_hbm.at[idx_all.at[pl.ds(0, CHUNK)]],
            rows[b].at[pl.ds(0, CHUNK)], sg[b]).wait()

    def start_wb(i, b):
      pltpu.async_copy(
          rows[b], out_hbm.at[pl.ds(base + i * GPB * CHUNK, GPB * CHUNK)],
          sw[b])

    def wait_wb(b):
      pltpu.make_async_copy(
          rows[b], out_hbm.at[pl.ds(base, GPB * CHUNK)], sw[b]).wait()

    for b in range(NBUF):
      start_gathers(b, b)

    def group(g, carry):
      for b in range(NBUF):
        wait_gathers(b)
        start_wb(g * NBUF + b, b)
      for b in range(NBUF):
        @pl.when(g + 1 < n_groups)
        def _():
          wait_wb(b)
          start_gathers((g + 1) * NBUF + b, b)
      return carry

    lax.fori_loop(0, n_groups, group, 0)
    for b in range(NBUF):
      wait_wb(b)

  return k(W, x_flat)


def kernel(x, W):
  batch, seq = x.shape
  D = W.shape[-1]
  B = batch * seq
  x_flat = x.reshape(B).astype(jnp.int32)
  out = _embedding_gather(x_flat, W, B, D)
  return out.reshape(batch, seq, D)


# speedup vs baseline: 1.0062x; 1.0062x over previous
import functools

import jax
import jax.numpy as jnp
from jax import lax
from jax.experimental import pallas as pl
from jax.experimental.pallas import tpu as pltpu
from jax.experimental.pallas import tpu_sc as plsc

NUM_CORES = 2
NUM_SUBCORES = 16
NUM_WORKERS = NUM_CORES * NUM_SUBCORES
CHUNK = 128
GPB = 1
NBUF = 5


@functools.partial(jax.jit, static_argnums=(2, 3))
def _embedding_gather(x_flat, W, B, D):
  b_per_w = B // NUM_WORKERS
  n_super = b_per_w // (CHUNK * GPB)
  n_groups = n_super // NBUF
  mesh = plsc.VectorSubcoreMesh(
      core_axis_name="c", subcore_axis_name="s",
      num_cores=NUM_CORES, num_subcores=NUM_SUBCORES)

  @functools.partial(
      pl.kernel,
      out_type=jax.ShapeDtypeStruct((B, D), jnp.float32),
      mesh=mesh,
      scratch_types=(
          [pltpu.VMEM((b_per_w,), jnp.int32)]
          + [pltpu.VMEM((GPB * CHUNK, D), jnp.float32) for _ in range(NBUF)]
          + [pltpu.SemaphoreType.DMA for _ in range(2 * NBUF)]
      ),
  )
  def k(table_hbm, idx_hbm, out_hbm, idx_all, *bufs_and_sems):
    rows = bufs_and_sems[:NBUF]
    sg = bufs_and_sems[NBUF:2 * NBUF]
    sw = bufs_and_sems[2 * NBUF:3 * NBUF]
    wid = lax.axis_index("s") * NUM_CORES + lax.axis_index("c")
    base = wid * b_per_w

    pltpu.sync_copy(idx_hbm.at[pl.ds(base, b_per_w)], idx_all)

    def start_gathers(i, b):
      for h in range(GPB):
        pltpu.async_copy(
            table_hbm.at[idx_all.at[pl.ds((i * GPB + h) * CHUNK, CHUNK)]],
            rows[b].at[pl.ds(h * CHUNK, CHUNK)], sg[b])

    def wait_gathers(b):
      for h in range(GPB):
        pltpu.make_async_copy(
            table_hbm.at[idx_all.at[pl.ds(0, CHUNK)]],
            rows[b].at[pl.ds(0, CHUNK)], sg[b]).wait()

    def start_wb(i, b):
      pltpu.async_copy(
          rows[b], out_hbm.at[pl.ds(base + i * GPB * CHUNK, GPB * CHUNK)],
          sw[b])

    def wait_wb(b):
      pltpu.make_async_copy(
          rows[b], out_hbm.at[pl.ds(base, GPB * CHUNK)], sw[b]).wait()

    for b in range(NBUF):
      start_gathers(b, b)

    def group(g, carry):
      for b in range(NBUF):
        wait_gathers(b)
        start_wb(g * NBUF + b, b)
      for b in range(NBUF):
        @pl.when(g + 1 < n_groups)
        def _():
          wait_wb(b)
          start_gathers((g + 1) * NBUF + b, b)
      return carry

    lax.fori_loop(0, n_groups, group, 0)
    for b in range(NBUF):
      wait_wb(b)

  return k(W, x_flat)


def kernel(x, W):
  batch, seq = x.shape
  D = W.shape[-1]
  B = batch * seq
  x_flat = x.reshape(B).astype(jnp.int32)
  out = _embedding_gather(x_flat, W, B, D)
  return out.reshape(batch, seq, D)
